# Initial kernel scaffold; baseline (speedup 1.0000x reference)
import jax
import jax.numpy as jnp
from jax.experimental import pallas as pl


def kernel(x, batch, edge_index, params, bn_stats):
    raise NotImplementedError("write your pallas kernel here")


# speedup vs baseline: 21.0567x; 21.0567x over previous
import functools

import jax
import jax.numpy as jnp
from jax import lax
from jax.experimental import pallas as pl
from jax.experimental.pallas import tpu as pltpu
from jax.experimental.pallas import tpu_sc as plsc

N = 102400
E = 3276800
NF = 16
DIM = 32
HALF = 16
NC = 2
NS = 16
KI = 8
ROWS128 = E // 128


def _make_sc_agg(split_edges: bool):
    mesh = plsc.VectorSubcoreMesh(core_axis_name="c", subcore_axis_name="s")
    rows_per_tile = ROWS128 // (NC * NS) if split_edges else ROWS128 // NS
    n_outer = rows_per_tile // KI
    rows_per_sub = N // NS

    @functools.partial(
        pl.kernel,
        mesh=mesh,
        compiler_params=pltpu.CompilerParams(use_tc_tiling_on_sc=False),
        out_type=jax.ShapeDtypeStruct((NC, N, HALF), jnp.float32),
        scratch_types=[
            pltpu.VMEM_SHARED((N, HALF), jnp.float32),
            pltpu.VMEM((KI, 128), jnp.int32),
            pltpu.VMEM((KI, 128), jnp.int32),
            pltpu.VMEM((KI, 128, HALF), jnp.float32),
            pltpu.SemaphoreType.DMA,
        ],
    )
    def k(h_hbm, src_hbm, dst_hbm, out_hbm, agg_s, idx_s, idx_d, rows_v, sem):
        c = lax.axis_index("c")
        s = lax.axis_index("s")
        hsrc = h_hbm if split_edges else h_hbm.at[c]
        pltpu.sync_copy(
            hsrc.at[pl.ds(s * rows_per_sub, rows_per_sub)],
            agg_s.at[pl.ds(s * rows_per_sub, rows_per_sub)],
        )
        plsc.subcore_barrier()
        tile_base = (
            (c * NS + s) * rows_per_tile if split_edges else s * rows_per_tile
        )

        def body(i, carry):
            base = tile_base + i * KI
            pltpu.sync_copy(src_hbm.at[pl.ds(base, KI)], idx_s)
            pltpu.sync_copy(dst_hbm.at[pl.ds(base, KI)], idx_d)
            cps = [
                pltpu.async_copy(hsrc.at[idx_s.at[j]], rows_v.at[j], sem)
                for j in range(KI)
            ]
            for cp in cps:
                cp.wait()
            for j in range(KI):
                pltpu.sync_copy(rows_v.at[j], agg_s.at[idx_d.at[j]], add=True)
            return carry

        lax.fori_loop(0, n_outer, body, 0)
        plsc.subcore_barrier()
        pltpu.sync_copy(
            agg_s.at[pl.ds(s * rows_per_sub, rows_per_sub)],
            out_hbm.at[c].at[pl.ds(s * rows_per_sub, rows_per_sub)],
        )

    return k


_sc_agg_l1 = _make_sc_agg(split_edges=True)
_sc_agg = _make_sc_agg(split_edges=False)


MLP_BLK = 4096


def _mlp_l1_body(p_ref, x_ref, w1, b1, w2, b2, sc, sh, out_ref):
    h2 = p_ref[0] + p_ref[1] - x_ref[...]
    a = jnp.maximum(
        jnp.dot(h2, w1[...], preferred_element_type=jnp.float32) + b1[...], 0.0
    )
    o = jnp.dot(a, w2[...], preferred_element_type=jnp.float32) + b2[...]
    o = jnp.maximum(o, 0.0) * sc[...] + sh[...]
    out_ref[0] = o[:, :HALF]
    out_ref[1] = o[:, HALF:]


def _mlp_body(s_ref, w1, b1, w2, b2, sc, sh, out_ref, *, out_split):
    h2 = jnp.concatenate([s_ref[0], s_ref[1]], axis=-1)
    a = jnp.maximum(
        jnp.dot(h2, w1[...], preferred_element_type=jnp.float32) + b1[...], 0.0
    )
    o = jnp.dot(a, w2[...], preferred_element_type=jnp.float32) + b2[...]
    o = jnp.maximum(o, 0.0) * sc[...] + sh[...]
    if out_split:
        out_ref[0] = o[:, :HALF]
        out_ref[1] = o[:, HALF:]
    else:
        out_ref[...] = o


def _vec_spec():
    return pl.BlockSpec((1, DIM), lambda i: (0, 0))


def _mlp_l1(p, x, w1, b1, w2, b2, sc, sh):
    grid = (N // MLP_BLK,)
    return pl.pallas_call(
        _mlp_l1_body,
        grid=grid,
        in_specs=[
            pl.BlockSpec((NC, MLP_BLK, HALF), lambda i: (0, i, 0)),
            pl.BlockSpec((MLP_BLK, HALF), lambda i: (i, 0)),
            pl.BlockSpec((HALF, DIM), lambda i: (0, 0)),
            _vec_spec(),
            pl.BlockSpec((DIM, DIM), lambda i: (0, 0)),
            _vec_spec(),
            _vec_spec(),
            _vec_spec(),
        ],
        out_specs=pl.BlockSpec((NC, MLP_BLK, HALF), lambda i: (0, i, 0)),
        out_shape=jax.ShapeDtypeStruct((NC, N, HALF), jnp.float32),
    )(p, x, w1, b1, w2, b2, sc, sh)


def _mlp(sagg, w1, b1, w2, b2, sc, sh, out_split):
    grid = (N // MLP_BLK,)
    if out_split:
        out_spec = pl.BlockSpec((NC, MLP_BLK, HALF), lambda i: (0, i, 0))
        out_shape = jax.ShapeDtypeStruct((NC, N, HALF), jnp.float32)
    else:
        out_spec = pl.BlockSpec((MLP_BLK, DIM), lambda i: (i, 0))
        out_shape = jax.ShapeDtypeStruct((N, DIM), jnp.float32)
    return pl.pallas_call(
        functools.partial(_mlp_body, out_split=out_split),
        grid=grid,
        in_specs=[
            pl.BlockSpec((NC, MLP_BLK, HALF), lambda i: (0, i, 0)),
            pl.BlockSpec((DIM, DIM), lambda i: (0, 0)),
            _vec_spec(),
            pl.BlockSpec((DIM, DIM), lambda i: (0, 0)),
            _vec_spec(),
            _vec_spec(),
            _vec_spec(),
        ],
        out_specs=out_spec,
        out_shape=out_shape,
    )(sagg, w1, b1, w2, b2, sc, sh)


POOL_BLK = 200
BATCH_N = N // 64


def _final_body(h_ref, s_ref, wf, bf, out_ref):
    pooled = jnp.dot(h_ref[...], s_ref[...], preferred_element_type=jnp.float32)
    logits = jnp.dot(pooled, wf[...], preferred_element_type=jnp.float32) + bf[...]
    m = jnp.max(logits, axis=-1, keepdims=True)
    e = jnp.exp(logits - m)
    out_ref[...] = logits - m - jnp.log(jnp.sum(e, axis=-1, keepdims=True))


def _final(hre, smat, wf, bf):
    grid = (BATCH_N // POOL_BLK,)
    return pl.pallas_call(
        _final_body,
        grid=grid,
        in_specs=[
            pl.BlockSpec((POOL_BLK, 64 * DIM), lambda i: (i, 0)),
            pl.BlockSpec((64 * DIM, DIM), lambda i: (0, 0)),
            pl.BlockSpec((DIM, 2), lambda i: (0, 0)),
            pl.BlockSpec((1, 2), lambda i: (0, 0)),
        ],
        out_specs=pl.BlockSpec((POOL_BLK, 2), lambda i: (i, 0)),
        out_shape=jax.ShapeDtypeStruct((BATCH_N, 2), jnp.float32),
    )(hre, smat, wf, bf)


def _bn_fold(bn_p, bn_s):
    scale = bn_p["gamma"] * lax.rsqrt(bn_s["var"] + 1e-5)
    shift = bn_p["beta"] - bn_s["mean"] * scale
    return scale.reshape(1, DIM), shift.reshape(1, DIM)


def kernel(x, batch, edge_index, params, bn_stats):
    del batch
    x = x.reshape(N, NF).astype(jnp.float32)
    src = edge_index[0].astype(jnp.int32).reshape(ROWS128, 128)
    dst = edge_index[1].astype(jnp.int32).reshape(ROWS128, 128)

    p1 = _sc_agg_l1(x, src, dst)
    w = params["nn1"]
    sc1, sh1 = _bn_fold(params["bn1"], bn_stats["bn1"])
    hcols = _mlp_l1(
        p1, x, w["W1"].T, w["b1"].reshape(1, DIM), w["W2"].T,
        w["b2"].reshape(1, DIM), sc1, sh1,
    )

    for i, out_split in ((2, True), (3, False)):
        sagg = _sc_agg(hcols, src, dst)
        w = params["nn%d" % i]
        sci, shi = _bn_fold(params["bn%d" % i], bn_stats["bn%d" % i])
        hcols = _mlp(
            sagg, w["W1"].T, w["b1"].reshape(1, DIM), w["W2"].T,
            w["b2"].reshape(1, DIM), sci, shi, out_split,
        )

    hre = hcols.reshape(BATCH_N, 64 * DIM)
    smat = jnp.tile(jnp.eye(DIM, dtype=jnp.float32), (64, 1))
    out = _final(
        hre, smat, params["fc2"]["W"].T, params["fc2"]["b"].reshape(1, 2)
    )
    return out
